# baseline (speedup 1.0000x reference)
import functools

import jax
import jax.numpy as jnp
from jax import lax
from jax.experimental import pallas as pl
from jax.experimental.pallas import tpu as pltpu
from jax.experimental.pallas import tpu_sc as plsc

EMB_DIM = 32
CHUNK = 128


@functools.lru_cache(maxsize=None)
def _make_gather(num_rows: int):
    info = plsc.get_sparse_core_info()
    nc, ns = info.num_cores, info.num_subcores
    nw = nc * ns
    n_chunks = num_rows // CHUNK
    k_per_w = n_chunks // nw
    assert n_chunks * CHUNK == num_rows and k_per_w * nw == n_chunks

    mesh = plsc.VectorSubcoreMesh(core_axis_name="c", subcore_axis_name="s")

    @functools.partial(
        pl.kernel,
        mesh=mesh,
        out_type=jax.ShapeDtypeStruct((num_rows, EMB_DIM), jnp.float32),
        compiler_params=pltpu.CompilerParams(use_tc_tiling_on_sc=False),
        scratch_types=[
            pltpu.VMEM((k_per_w * CHUNK,), jnp.int32),
            pltpu.VMEM((CHUNK, EMB_DIM), jnp.float32),
            pltpu.SemaphoreType.DMA,
        ],
    )
    def gather_kernel(idx_hbm, table_hbm, out_hbm, idx_v, buf, sem):
        wid = lax.axis_index("s") * nc + lax.axis_index("c")
        base = wid * (k_per_w * CHUNK)
        pltpu.sync_copy(idx_hbm.at[pl.ds(base, k_per_w * CHUNK)], idx_v)

        def body(j, carry):
            idx_c = idx_v.at[pl.ds(j * CHUNK, CHUNK)]
            pltpu.async_copy(table_hbm.at[idx_c], buf, sem).wait()
            pltpu.sync_copy(buf, out_hbm.at[pl.ds(base + j * CHUNK, CHUNK)])
            return carry

        lax.fori_loop(0, k_per_w, body, 0)

    return gather_kernel


def kernel(indices, table):
    b, h = indices.shape
    num_rows = b * h
    idx_flat = indices.reshape(num_rows).astype(jnp.int32)
    out = _make_gather(num_rows)(idx_flat, table)
    return out.reshape(b, h, EMB_DIM)


# speedup vs baseline: 1.0448x; 1.0448x over previous
import functools

import jax
import jax.numpy as jnp
from jax import lax
from jax.experimental import pallas as pl
from jax.experimental.pallas import tpu as pltpu
from jax.experimental.pallas import tpu_sc as plsc

EMB_DIM = 32
CHUNK = 1600


@functools.lru_cache(maxsize=None)
def _make_gather(num_rows: int):
    info = plsc.get_sparse_core_info()
    nc, ns = info.num_cores, info.num_subcores
    nw = nc * ns
    n_chunks = num_rows // CHUNK
    k_per_w = n_chunks // nw
    assert n_chunks * CHUNK == num_rows and k_per_w * nw == n_chunks

    mesh = plsc.VectorSubcoreMesh(core_axis_name="c", subcore_axis_name="s")

    @functools.partial(
        pl.kernel,
        mesh=mesh,
        out_type=jax.ShapeDtypeStruct((num_rows, EMB_DIM), jnp.float32),
        compiler_params=pltpu.CompilerParams(use_tc_tiling_on_sc=False),
        scratch_types=[
            pltpu.VMEM((k_per_w * CHUNK,), jnp.int32),
            pltpu.VMEM((CHUNK, EMB_DIM), jnp.float32),
            pltpu.SemaphoreType.DMA,
        ],
    )
    def gather_kernel(idx_hbm, table_hbm, out_hbm, idx_v, buf, sem):
        wid = lax.axis_index("s") * nc + lax.axis_index("c")
        base = wid * (k_per_w * CHUNK)
        pltpu.sync_copy(idx_hbm.at[pl.ds(base, k_per_w * CHUNK)], idx_v)

        def body(j, carry):
            idx_c = idx_v.at[pl.ds(j * CHUNK, CHUNK)]
            pltpu.async_copy(table_hbm.at[idx_c], buf, sem).wait()
            pltpu.sync_copy(buf, out_hbm.at[pl.ds(base + j * CHUNK, CHUNK)])
            return carry

        lax.fori_loop(0, k_per_w, body, 0)

    return gather_kernel


def kernel(indices, table):
    b, h = indices.shape
    num_rows = b * h
    idx_flat = indices.reshape(num_rows).astype(jnp.int32)
    out = _make_gather(num_rows)(idx_flat, table)
    return out.reshape(b, h, EMB_DIM)
